# baseline (speedup 1.0000x reference)
import functools

import jax
import jax.numpy as jnp
from jax import lax
from jax.experimental import pallas as pl
from jax.experimental.pallas import tpu as pltpu
from jax.experimental.pallas import tpu_sc as plsc

_K = 1024
_D = 64
_T = 1024


def _dist_argmin_body(x_ref, e_ref, ind_ref):
    f = x_ref[...]
    e = e_ref[...]
    f2 = jnp.sum(f * f, axis=1, keepdims=True)
    mm = lax.dot_general(2.0 * f, e, (((1,), (1,)), ((), ())),
                         preferred_element_type=jnp.float32)
    e2 = jnp.sum(e * e, axis=1)[None, :]
    dist = -(f2 - mm + e2)
    mx = jnp.max(dist, axis=1, keepdims=True)
    col = lax.broadcasted_iota(jnp.int32, dist.shape, 1)
    ind = jnp.min(jnp.where(dist == mx, col, _K), axis=1)
    ind_ref[...] = ind.astype(jnp.int32)


def _tc_argmin(flat, embed):
    n = flat.shape[0]
    grid = n // _T
    return pl.pallas_call(
        _dist_argmin_body,
        grid=(grid,),
        in_specs=[
            pl.BlockSpec((_T, _D), lambda i: (i, 0)),
            pl.BlockSpec((_K, _D), lambda i: (0, 0)),
        ],
        out_specs=pl.BlockSpec((_T,), lambda i: (i,)),
        out_shape=jax.ShapeDtypeStruct((n,), jnp.int32),
    )(flat, embed)


_NC = 2
_NS = 16
_NW = _NC * _NS
_BPW = 9216 // _NW
_CH = 96
_NCH = _BPW // _CH


@functools.cache
def _make_sc_gather():
    @functools.partial(
        pl.kernel,
        mesh=plsc.VectorSubcoreMesh(core_axis_name="c", subcore_axis_name="s"),
        out_type=jax.ShapeDtypeStruct((9216, _D), jnp.float32),
        compiler_params=pltpu.CompilerParams(use_tc_tiling_on_sc=False),
        scratch_types=[
            pltpu.VMEM((_NCH, _CH), jnp.int32),
            pltpu.VMEM((_BPW, _D), jnp.float32),
            pltpu.SemaphoreType.DMA,
        ],
    )
    def _sc_gather(embed_hbm, idx_hbm, out_hbm, idx_v, rows_v, sem):
        wid = lax.axis_index("s") * _NC + lax.axis_index("c")
        base = wid * _BPW
        pltpu.sync_copy(idx_hbm.at[wid], idx_v)
        copies = [pltpu.async_copy(embed_hbm.at[idx_v.at[j]],
                                   rows_v.at[pl.ds(j * _CH, _CH)], sem)
                  for j in range(_NCH)]
        for c in copies:
            c.wait()
        pltpu.sync_copy(rows_v, out_hbm.at[pl.ds(base, _BPW)])

    return _sc_gather


def kernel(x, embed):
    shape = x.shape
    flat = x.reshape(-1, shape[-1])
    ind = _tc_argmin(flat, embed)
    idx3 = ind.reshape(_NW, _NCH, _CH)
    quant = _make_sc_gather()(embed, idx3)
    return quant.reshape(shape), ind.reshape(shape[:-1])


# speedup vs baseline: 1.0861x; 1.0861x over previous
import functools

import jax
import jax.numpy as jnp
from jax import lax
from jax.experimental import pallas as pl
from jax.experimental.pallas import tpu as pltpu
from jax.experimental.pallas import tpu_sc as plsc

_K = 1024
_D = 64
_T = 1024


def _dist_argmin_body(x_ref, e_ref, ind_ref):
    f = x_ref[...]
    e = e_ref[...]
    ft = f.T
    f2 = jnp.sum(ft * ft, axis=0, keepdims=True).T
    et = e.T
    e2 = jnp.sum(et * et, axis=0, keepdims=True)
    mm = lax.dot_general(2.0 * f, e, (((1,), (1,)), ((), ())),
                         preferred_element_type=jnp.float32)
    dist = -(f2 - mm + e2)
    mx = jnp.max(dist, axis=1, keepdims=True)
    col = lax.broadcasted_iota(jnp.int32, dist.shape, 1)
    ind = jnp.min(jnp.where(dist == mx, col, _K), axis=1)
    ind_ref[...] = ind.astype(jnp.int32)


def _tc_argmin(flat, embed):
    n = flat.shape[0]
    grid = n // _T
    return pl.pallas_call(
        _dist_argmin_body,
        grid=(grid,),
        in_specs=[
            pl.BlockSpec((_T, _D), lambda i: (i, 0)),
            pl.BlockSpec((_K, _D), lambda i: (0, 0)),
        ],
        out_specs=pl.BlockSpec((_T,), lambda i: (i,)),
        out_shape=jax.ShapeDtypeStruct((n,), jnp.int32),
    )(flat, embed)


_NC = 2
_NS = 16
_NW = _NC * _NS
_BPW = 9216 // _NW
_CH = 96
_NCH = _BPW // _CH


@functools.cache
def _make_sc_gather():
    @functools.partial(
        pl.kernel,
        mesh=plsc.VectorSubcoreMesh(core_axis_name="c", subcore_axis_name="s"),
        out_type=jax.ShapeDtypeStruct((9216, _D), jnp.float32),
        compiler_params=pltpu.CompilerParams(use_tc_tiling_on_sc=False),
        scratch_types=[
            pltpu.VMEM((_NCH, _CH), jnp.int32),
            pltpu.VMEM((_BPW, _D), jnp.float32),
            pltpu.SemaphoreType.DMA,
        ],
    )
    def _sc_gather(embed_hbm, idx_hbm, out_hbm, idx_v, rows_v, sem):
        wid = lax.axis_index("s") * _NC + lax.axis_index("c")
        base = wid * _BPW
        pltpu.sync_copy(idx_hbm.at[wid], idx_v)
        copies = [pltpu.async_copy(embed_hbm.at[idx_v.at[j]],
                                   rows_v.at[pl.ds(j * _CH, _CH)], sem)
                  for j in range(_NCH)]
        for c in copies:
            c.wait()
        pltpu.sync_copy(rows_v, out_hbm.at[pl.ds(base, _BPW)])

    return _sc_gather


def kernel(x, embed):
    shape = x.shape
    flat = x.reshape(-1, shape[-1])
    ind = _tc_argmin(flat, embed)
    idx3 = ind.reshape(_NW, _NCH, _CH)
    quant = _make_sc_gather()(embed, idx3)
    return quant.reshape(shape), ind.reshape(shape[:-1])


# speedup vs baseline: 1.1522x; 1.0608x over previous
import functools

import jax
import jax.numpy as jnp
from jax import lax
from jax.experimental import pallas as pl
from jax.experimental.pallas import tpu as pltpu
from jax.experimental.pallas import tpu_sc as plsc

_K = 1024
_D = 64
_T = 3072


def _dist_argmin_body(x_ref, e_ref, ind_ref, e2_ref):
    @pl.when(pl.program_id(0) == 0)
    def _():
        et = e_ref[...].T
        e2_ref[...] = jnp.sum(et * et, axis=0, keepdims=True)

    f = x_ref[...]
    ft = f.T
    f2 = jnp.sum(ft * ft, axis=0, keepdims=True).T
    mm = lax.dot_general(2.0 * f, e_ref[...], (((1,), (1,)), ((), ())),
                         preferred_element_type=jnp.float32)
    d = f2 - mm + e2_ref[...]
    mn = jnp.min(d, axis=1, keepdims=True)
    col = lax.broadcasted_iota(jnp.int32, d.shape, 1)
    ind = jnp.min(jnp.where(d == mn, col, _K), axis=1)
    ind_ref[...] = ind.astype(jnp.int32)


def _tc_argmin(flat, embed):
    n = flat.shape[0]
    grid = n // _T
    return pl.pallas_call(
        _dist_argmin_body,
        grid=(grid,),
        in_specs=[
            pl.BlockSpec((_T, _D), lambda i: (i, 0)),
            pl.BlockSpec((_K, _D), lambda i: (0, 0)),
        ],
        out_specs=pl.BlockSpec((_T,), lambda i: (i,)),
        out_shape=jax.ShapeDtypeStruct((n,), jnp.int32),
        scratch_shapes=[pltpu.VMEM((1, _K), jnp.float32)],
    )(flat, embed)


_NC = 2
_NS = 16
_NW = _NC * _NS
_BPW = 9216 // _NW
_CH = 96
_NCH = _BPW // _CH


@functools.cache
def _make_sc_gather():
    @functools.partial(
        pl.kernel,
        mesh=plsc.VectorSubcoreMesh(core_axis_name="c", subcore_axis_name="s"),
        out_type=jax.ShapeDtypeStruct((9216, _D), jnp.float32),
        compiler_params=pltpu.CompilerParams(use_tc_tiling_on_sc=False),
        scratch_types=[
            pltpu.VMEM((_NCH, _CH), jnp.int32),
            pltpu.VMEM((_BPW, _D), jnp.float32),
            pltpu.SemaphoreType.DMA,
        ],
    )
    def _sc_gather(embed_hbm, idx_hbm, out_hbm, idx_v, rows_v, sem):
        wid = lax.axis_index("s") * _NC + lax.axis_index("c")
        base = wid * _BPW
        pltpu.sync_copy(idx_hbm.at[wid], idx_v)
        copies = [pltpu.async_copy(embed_hbm.at[idx_v.at[j]],
                                   rows_v.at[pl.ds(j * _CH, _CH)], sem)
                  for j in range(_NCH)]
        for c in copies:
            c.wait()
        pltpu.sync_copy(rows_v, out_hbm.at[pl.ds(base, _BPW)])

    return _sc_gather


def kernel(x, embed):
    shape = x.shape
    flat = x.reshape(-1, shape[-1])
    ind = _tc_argmin(flat, embed)
    idx3 = ind.reshape(_NW, _NCH, _CH)
    quant = _make_sc_gather()(embed, idx3)
    return quant.reshape(shape), ind.reshape(shape[:-1])


# speedup vs baseline: 1.1870x; 1.0303x over previous
import functools

import jax
import jax.numpy as jnp
from jax import lax
from jax.experimental import pallas as pl
from jax.experimental.pallas import tpu as pltpu
from jax.experimental.pallas import tpu_sc as plsc

_K = 1024
_D = 64
_T = 3072


def _dist_argmin_body(x_ref, e_ref, ind_ref, e2_ref):
    @pl.when(pl.program_id(0) == 0)
    def _():
        et = e_ref[...].T
        e2_ref[...] = jnp.sum(et * et, axis=0, keepdims=True)

    f = x_ref[...]
    ft = f.T
    f2 = jnp.sum(ft * ft, axis=0, keepdims=True).T
    mm = lax.dot_general(2.0 * f, e_ref[...], (((1,), (1,)), ((), ())),
                         preferred_element_type=jnp.float32)
    d = f2 - mm + e2_ref[...]
    ind_ref[...] = jnp.argmin(d, axis=1).astype(jnp.int32)


def _tc_argmin(flat, embed):
    n = flat.shape[0]
    grid = n // _T
    return pl.pallas_call(
        _dist_argmin_body,
        grid=(grid,),
        in_specs=[
            pl.BlockSpec((_T, _D), lambda i: (i, 0)),
            pl.BlockSpec((_K, _D), lambda i: (0, 0)),
        ],
        out_specs=pl.BlockSpec((_T,), lambda i: (i,)),
        out_shape=jax.ShapeDtypeStruct((n,), jnp.int32),
        scratch_shapes=[pltpu.VMEM((1, _K), jnp.float32)],
    )(flat, embed)


_NC = 2
_NS = 16
_NW = _NC * _NS
_BPW = 9216 // _NW
_CH = 96
_NCH = _BPW // _CH


@functools.cache
def _make_sc_gather():
    @functools.partial(
        pl.kernel,
        mesh=plsc.VectorSubcoreMesh(core_axis_name="c", subcore_axis_name="s"),
        out_type=jax.ShapeDtypeStruct((9216, _D), jnp.float32),
        compiler_params=pltpu.CompilerParams(use_tc_tiling_on_sc=False),
        scratch_types=[
            pltpu.VMEM((_NCH, _CH), jnp.int32),
            pltpu.VMEM((_BPW, _D), jnp.float32),
            pltpu.SemaphoreType.DMA,
        ],
    )
    def _sc_gather(embed_hbm, idx_hbm, out_hbm, idx_v, rows_v, sem):
        wid = lax.axis_index("s") * _NC + lax.axis_index("c")
        base = wid * _BPW
        pltpu.sync_copy(idx_hbm.at[wid], idx_v)
        copies = [pltpu.async_copy(embed_hbm.at[idx_v.at[j]],
                                   rows_v.at[pl.ds(j * _CH, _CH)], sem)
                  for j in range(_NCH)]
        for c in copies:
            c.wait()
        pltpu.sync_copy(rows_v, out_hbm.at[pl.ds(base, _BPW)])

    return _sc_gather


def kernel(x, embed):
    shape = x.shape
    flat = x.reshape(-1, shape[-1])
    ind = _tc_argmin(flat, embed)
    idx3 = ind.reshape(_NW, _NCH, _CH)
    quant = _make_sc_gather()(embed, idx3)
    return quant.reshape(shape), ind.reshape(shape[:-1])
